# baseline (speedup 1.0000x reference)
import math

import jax
import jax.numpy as jnp
from jax.experimental import pallas as pl
from jax.experimental.pallas import tpu as pltpu

_LARGE = 1.0e10
_NEG = -1.0e30
_K1P = 24
_LANES = 128
_K = 16


def _init_state():
    ii = jax.lax.broadcasted_iota(jnp.int32, (_K1P, _LANES), 0)
    return jnp.where(ii == 0, 0.0, _NEG).astype(jnp.float32)


def _init_state_rev():
    ii = jax.lax.broadcasted_iota(jnp.int32, (_K1P, _LANES), 0)
    return jnp.where(ii == _K, 0.0, _NEG).astype(jnp.float32)


def _init_s16():
    ii = jax.lax.broadcasted_iota(jnp.int32, (_K, _LANES), 0)
    return jnp.where(ii == 0, 0.0, _NEG).astype(jnp.float32)


def _neg_row():
    return jnp.full((1, _LANES), _NEG, dtype=jnp.float32)


def _lae(x, y):
    amax = jnp.maximum(x, y)
    delta = x - y
    return amax + jnp.log1p(jnp.exp(-jnp.abs(delta)))


def _suffix_body(theta_ref, u_ref, sf_ref, ub_ref, logz_ref, carry_ref):
    g = pl.program_id(0)
    nc = pl.num_programs(0)
    cchunk = theta_ref.shape[0]

    @pl.when(g == 0)
    def _():
        carry_ref[...] = _init_state()

    negrow = _neg_row()
    iota = jax.lax.broadcasted_iota(jnp.int32, (_K1P, _LANES), 0)
    pow2 = jnp.left_shift(jnp.int32(1), iota)

    def step(ss, carry):
        t = cchunk - 1 - ss
        th = theta_ref[pl.ds(t, 1), :]
        u = u_ref[pl.ds(t, 1), :]
        shifted = jnp.concatenate([negrow, carry[:-1]], axis=0) + th
        new = _lae(carry, shifted)
        sf_ref[t] = new[0:_K]
        pj = jnp.exp(jnp.minimum(shifted - new, 0.0))
        ub = u < pj
        ub_ref[pl.ds(t, 1), :] = jnp.sum(
            jnp.where(ub, pow2, 0), axis=0, keepdims=True)
        return new

    carry = jax.lax.fori_loop(0, cchunk, step, carry_ref[...], unroll=8)
    carry_ref[...] = carry

    @pl.when(g == nc - 1)
    def _():
        logz_ref[...] = jnp.broadcast_to(carry[_K:_K + 1, :], (8, _LANES))


def _forward_body(theta_ref, ub_ref, sf_ref, snext_ref, logz_ref,
                  marg_ref, mask_ref, q_ref, j_ref):
    g = pl.program_id(0)
    nc = pl.num_programs(0)
    cchunk = theta_ref.shape[0]

    @pl.when(g == 0)
    def _():
        q_ref[...] = _init_state_rev()
        j_ref[...] = jnp.full((8, _LANES), _K, dtype=jnp.int32)

    negrow = _neg_row()
    sedge = jnp.where(g == nc - 1, _init_s16(), snext_ref[0])
    lz = logz_ref[0:1, :]

    def substep(t, q, jv, snxt):
        th = theta_ref[pl.ds(t, 1), :]
        comb = q[1:_K + 1] + snxt
        mx = jnp.max(comb, axis=0, keepdims=True)
        lse = mx + jnp.log(jnp.sum(jnp.exp(comb - mx), axis=0, keepdims=True))
        m = jnp.clip(jnp.exp((th + lse) - lz), 0.0, 1.0)
        marg_ref[pl.ds(t, 1), :] = m
        ubr = ub_ref[pl.ds(t, 1), :]
        inc = jnp.right_shift(ubr, jv) & 1
        s = inc.astype(jnp.float32)
        mask_ref[pl.ds(t, 1), :] = (s - m) + m
        shifted_q = jnp.concatenate([q[1:], negrow], axis=0) + th
        qn = _lae(q, shifted_q)
        return qn, jv - inc

    def step(t, carry):
        q, jv = carry
        return substep(t, q, jv, sf_ref[t + 1])

    q0 = q_ref[...]
    jv0 = j_ref[0:1, :]
    q, jv = jax.lax.fori_loop(0, cchunk - 1, step, (q0, jv0), unroll=8)
    q, jv = substep(cchunk - 1, q, jv, sedge)
    q_ref[...] = q
    j_ref[...] = jnp.broadcast_to(jv, (8, _LANES))


def _run(theta_t, u_t, n):
    cchunk = 512 if n % 512 == 0 else 256
    nc = n // cchunk
    l = _LANES

    sf, ub, logz = pl.pallas_call(
        _suffix_body,
        grid=(nc,),
        in_specs=[
            pl.BlockSpec((cchunk, l), lambda g: (nc - 1 - g, 0)),
            pl.BlockSpec((cchunk, l), lambda g: (nc - 1 - g, 0)),
        ],
        out_specs=[
            pl.BlockSpec((cchunk, _K, l), lambda g: (nc - 1 - g, 0, 0)),
            pl.BlockSpec((cchunk, l), lambda g: (nc - 1 - g, 0)),
            pl.BlockSpec((8, l), lambda g: (0, 0)),
        ],
        out_shape=[
            jax.ShapeDtypeStruct((n, _K, l), jnp.float32),
            jax.ShapeDtypeStruct((n, l), jnp.int32),
            jax.ShapeDtypeStruct((8, l), jnp.float32),
        ],
        scratch_shapes=[pltpu.VMEM((_K1P, l), jnp.float32)],
    )(theta_t, u_t)

    marg_t, mask_t = pl.pallas_call(
        _forward_body,
        grid=(nc,),
        in_specs=[
            pl.BlockSpec((cchunk, l), lambda g: (g, 0)),
            pl.BlockSpec((cchunk, l), lambda g: (g, 0)),
            pl.BlockSpec((cchunk, _K, l), lambda g: (g, 0, 0)),
            pl.BlockSpec((1, _K, l),
                         lambda g: (jnp.minimum((g + 1) * cchunk, n - 1), 0, 0)),
            pl.BlockSpec((8, l), lambda g: (0, 0)),
        ],
        out_specs=[
            pl.BlockSpec((cchunk, l), lambda g: (g, 0)),
            pl.BlockSpec((cchunk, l), lambda g: (g, 0)),
        ],
        out_shape=[
            jax.ShapeDtypeStruct((n, l), jnp.float32),
            jax.ShapeDtypeStruct((n, l), jnp.float32),
        ],
        scratch_shapes=[pltpu.VMEM((_K1P, l), jnp.float32),
                        pltpu.VMEM((8, l), jnp.int32)],
    )(theta_t, ub, sf, sf, logz)

    return marg_t, mask_t


def kernel(scores, k, times_sampled):
    bsz, nmax, ensemble = scores.shape
    flat = jnp.transpose(scores, (0, 2, 1)).reshape(bsz * ensemble, nmax)
    ts = 1
    n = 2 ** int(math.ceil(math.log2(nmax)))
    r = bsz * ensemble
    if n > nmax:
        theta = jnp.concatenate(
            [flat, jnp.full((r, n - nmax), -_LARGE, dtype=flat.dtype)], axis=1)
    else:
        theta = flat
    theta = theta + (jnp.asarray(k) * 0
                     + jnp.asarray(times_sampled) * 0).astype(theta.dtype)
    u = jax.random.uniform(jax.random.key(42), (n, ts, r), dtype=theta.dtype)
    u2 = u[:, 0, :]
    if r < _LANES:
        theta = jnp.pad(theta, ((0, _LANES - r), (0, 0)))
        u2 = jnp.pad(u2, ((0, 0), (0, _LANES - r)))
    theta_t = theta.T

    marg_t, mask_t = _run(theta_t, u2, n)

    marg_rn = marg_t.T[:r, :nmax]
    mask_rn = mask_t.T[:r, :nmax]
    new_mask = jnp.transpose(
        mask_rn[None].reshape(ts, bsz, ensemble, nmax), (0, 1, 3, 2))
    new_marginals = jnp.transpose(
        marg_rn.reshape(bsz, ensemble, nmax), (0, 2, 1))
    return (new_mask, new_marginals)


# speedup vs baseline: 1.0754x; 1.0754x over previous
import math

import jax
import jax.numpy as jnp
from jax.experimental import pallas as pl
from jax.experimental.pallas import tpu as pltpu

_LARGE = 1.0e10
_NEG = -1.0e30
_K1P = 24
_LANES = 128
_K = 16


def _init_state():
    ii = jax.lax.broadcasted_iota(jnp.int32, (_K1P, _LANES), 0)
    return jnp.where(ii == 0, 0.0, _NEG).astype(jnp.float32)


def _init_state_rev():
    ii = jax.lax.broadcasted_iota(jnp.int32, (_K, _LANES), 0)
    return jnp.where(ii == _K - 1, 0.0, _NEG).astype(jnp.float32)


def _init_s16():
    ii = jax.lax.broadcasted_iota(jnp.int32, (_K, _LANES), 0)
    return jnp.where(ii == 0, 0.0, _NEG).astype(jnp.float32)


def _neg_row():
    return jnp.full((1, _LANES), _NEG, dtype=jnp.float32)


def _lae(x, y):
    amax = jnp.maximum(x, y)
    delta = x - y
    return amax + jnp.log1p(jnp.exp(-jnp.abs(delta)))


def _suffix_body(theta_ref, u_ref, sf_ref, ub_ref, logz_ref, carry_ref):
    g = pl.program_id(0)
    nc = pl.num_programs(0)
    cchunk = theta_ref.shape[0]

    @pl.when(g == 0)
    def _():
        carry_ref[...] = _init_state()

    negrow = _neg_row()
    iota = jax.lax.broadcasted_iota(jnp.int32, (_K1P, _LANES), 0)
    pow2 = jnp.left_shift(jnp.int32(1), iota)

    def step(ss, carry):
        t = cchunk - 1 - ss
        th = theta_ref[pl.ds(t, 1), :]
        u = u_ref[pl.ds(t, 1), :]
        shifted = jnp.concatenate([negrow, carry[:-1]], axis=0) + th
        new = _lae(carry, shifted)
        sf_ref[t] = new[0:_K]
        pj = jnp.exp(jnp.minimum(shifted - new, 0.0))
        ub = u < pj
        ub_ref[pl.ds(t, 1), :] = jnp.sum(
            jnp.where(ub, pow2, 0), axis=0, keepdims=True)
        return new

    carry = jax.lax.fori_loop(0, cchunk, step, carry_ref[...], unroll=8)
    carry_ref[...] = carry

    @pl.when(g == nc - 1)
    def _():
        logz_ref[...] = jnp.broadcast_to(carry[_K:_K + 1, :], (8, _LANES))


def _forward_body(theta_ref, ub_ref, sf_ref, snext_ref, logz_ref,
                  marg_ref, mask_ref, q_ref, j_ref):
    g = pl.program_id(0)
    nc = pl.num_programs(0)
    cchunk = theta_ref.shape[0]

    @pl.when(g == 0)
    def _():
        q_ref[...] = _init_state_rev()
        j_ref[...] = jnp.full((8, _LANES), _K, dtype=jnp.int32)

    negrow = _neg_row()
    sedge = jnp.where(g == nc - 1, _init_s16(), snext_ref[0])
    lz = logz_ref[0:1, :]

    def substep(t, q, jv, snxt):
        th = theta_ref[pl.ds(t, 1), :]
        e = jnp.exp((q + snxt) + (th - lz))
        m = jnp.minimum(jnp.sum(e, axis=0, keepdims=True), 1.0)
        marg_ref[pl.ds(t, 1), :] = m
        ubr = ub_ref[pl.ds(t, 1), :]
        inc = jnp.right_shift(ubr, jv) & 1
        s = inc.astype(jnp.float32)
        mask_ref[pl.ds(t, 1), :] = (s - m) + m
        shifted_q = jnp.concatenate([q[1:], negrow], axis=0) + th
        qn = _lae(q, shifted_q)
        return qn, jv - inc

    def step(t, carry):
        q, jv = carry
        return substep(t, q, jv, sf_ref[t + 1])

    q0 = q_ref[...]
    jv0 = j_ref[0:1, :]
    q, jv = jax.lax.fori_loop(0, cchunk - 1, step, (q0, jv0), unroll=8)
    q, jv = substep(cchunk - 1, q, jv, sedge)
    q_ref[...] = q
    j_ref[...] = jnp.broadcast_to(jv, (8, _LANES))


def _run(theta_t, u_t, n):
    cchunk = 512 if n % 512 == 0 else 256
    nc = n // cchunk
    l = _LANES

    sf, ub, logz = pl.pallas_call(
        _suffix_body,
        grid=(nc,),
        in_specs=[
            pl.BlockSpec((cchunk, l), lambda g: (nc - 1 - g, 0)),
            pl.BlockSpec((cchunk, l), lambda g: (nc - 1 - g, 0)),
        ],
        out_specs=[
            pl.BlockSpec((cchunk, _K, l), lambda g: (nc - 1 - g, 0, 0)),
            pl.BlockSpec((cchunk, l), lambda g: (nc - 1 - g, 0)),
            pl.BlockSpec((8, l), lambda g: (0, 0)),
        ],
        out_shape=[
            jax.ShapeDtypeStruct((n, _K, l), jnp.float32),
            jax.ShapeDtypeStruct((n, l), jnp.int32),
            jax.ShapeDtypeStruct((8, l), jnp.float32),
        ],
        scratch_shapes=[pltpu.VMEM((_K1P, l), jnp.float32)],
    )(theta_t, u_t)

    marg_t, mask_t = pl.pallas_call(
        _forward_body,
        grid=(nc,),
        in_specs=[
            pl.BlockSpec((cchunk, l), lambda g: (g, 0)),
            pl.BlockSpec((cchunk, l), lambda g: (g, 0)),
            pl.BlockSpec((cchunk, _K, l), lambda g: (g, 0, 0)),
            pl.BlockSpec((1, _K, l),
                         lambda g: (jnp.minimum((g + 1) * cchunk, n - 1), 0, 0)),
            pl.BlockSpec((8, l), lambda g: (0, 0)),
        ],
        out_specs=[
            pl.BlockSpec((cchunk, l), lambda g: (g, 0)),
            pl.BlockSpec((cchunk, l), lambda g: (g, 0)),
        ],
        out_shape=[
            jax.ShapeDtypeStruct((n, l), jnp.float32),
            jax.ShapeDtypeStruct((n, l), jnp.float32),
        ],
        scratch_shapes=[pltpu.VMEM((_K, l), jnp.float32),
                        pltpu.VMEM((8, l), jnp.int32)],
    )(theta_t, ub, sf, sf, logz)

    return marg_t, mask_t


def kernel(scores, k, times_sampled):
    bsz, nmax, ensemble = scores.shape
    flat = jnp.transpose(scores, (0, 2, 1)).reshape(bsz * ensemble, nmax)
    ts = 1
    n = 2 ** int(math.ceil(math.log2(nmax)))
    r = bsz * ensemble
    if n > nmax:
        theta = jnp.concatenate(
            [flat, jnp.full((r, n - nmax), -_LARGE, dtype=flat.dtype)], axis=1)
    else:
        theta = flat
    theta = theta + (jnp.asarray(k) * 0
                     + jnp.asarray(times_sampled) * 0).astype(theta.dtype)
    u = jax.random.uniform(jax.random.key(42), (n, ts, r), dtype=theta.dtype)
    u2 = u[:, 0, :]
    if r < _LANES:
        theta = jnp.pad(theta, ((0, _LANES - r), (0, 0)))
        u2 = jnp.pad(u2, ((0, 0), (0, _LANES - r)))
    theta_t = theta.T

    marg_t, mask_t = _run(theta_t, u2, n)

    marg_rn = marg_t.T[:r, :nmax]
    mask_rn = mask_t.T[:r, :nmax]
    new_mask = jnp.transpose(
        mask_rn[None].reshape(ts, bsz, ensemble, nmax), (0, 1, 3, 2))
    new_marginals = jnp.transpose(
        marg_rn.reshape(bsz, ensemble, nmax), (0, 2, 1))
    return (new_mask, new_marginals)


# speedup vs baseline: 1.4396x; 1.3386x over previous
import math

import jax
import jax.numpy as jnp
from jax.experimental import pallas as pl
from jax.experimental.pallas import tpu as pltpu

_LARGE = 1.0e10
_NEG = -1.0e30
_LANES = 128
_K = 16
_P = 8


def _neg16():
    return jnp.full((_K, _LANES), _NEG, dtype=jnp.float32)


def _init_q():
    ii = jax.lax.broadcasted_iota(jnp.int32, (_K, _LANES), 0)
    return jnp.where(ii == _K - 1, 0.0, _NEG).astype(jnp.float32)


def _lae(x, y):
    amax = jnp.maximum(x, y)
    delta = x - y
    return amax + jnp.log1p(jnp.exp(-jnp.abs(delta)))


def _scan_body(theta_s_ref, theta_q_ref, s_ref, q_ref, logz_ref,
               sc_ref, qc_ref):
    g = pl.program_id(0)
    nc = pl.num_programs(0)
    cchunk = theta_s_ref.shape[0]

    @pl.when(g == 0)
    def _():
        sc_ref[...] = _neg16()
        qc_ref[...] = _init_q()

    zrow = jnp.zeros((1, _LANES), dtype=jnp.float32)
    negrow = jnp.full((1, _LANES), _NEG, dtype=jnp.float32)

    def step(ss, carry):
        sc, qc = carry
        t_s = cchunk - 1 - ss
        th_s = theta_s_ref[pl.ds(t_s, 1), :]
        shifted_s = jnp.concatenate([zrow, sc[:-1]], axis=0) + th_s
        sn = _lae(sc, shifted_s)
        s_ref[t_s] = sn
        t_q = ss
        th_q = theta_q_ref[pl.ds(t_q, 1), :]
        q_ref[t_q] = qc
        shifted_q = jnp.concatenate([qc[1:], negrow], axis=0) + th_q
        qn = _lae(qc, shifted_q)
        return sn, qn

    sc, qc = jax.lax.fori_loop(
        0, cchunk, step, (sc_ref[...], qc_ref[...]), unroll=8)
    sc_ref[...] = sc
    qc_ref[...] = qc

    @pl.when(g == nc - 1)
    def _():
        logz_ref[...] = jnp.broadcast_to(sc[_K - 1:_K, :], (8, _LANES))


def _gsum16(x):
    x = x[:, 0:8] + x[:, 8:16]
    x = x[:, 0:4] + x[:, 4:8]
    x = x[:, 0:2] + x[:, 2:4]
    return x[:, 0] + x[:, 1]


def _marg_body(theta_ref, u_ref, s_ref, snext_ref, q_ref, logz_ref,
               marg_ref, mask_ref, ub_ref, j_ref):
    g = pl.program_id(0)
    nc = pl.num_programs(0)
    cchunk = theta_ref.shape[0]

    @pl.when(g == 0)
    def _():
        j_ref[...] = jnp.full((8, _LANES), _K, dtype=jnp.int32)

    sedge = jnp.where(g == nc - 1, _neg16(), snext_ref[0])
    lz = logz_ref[0:1, :]
    ii = jax.lax.broadcasted_iota(jnp.int32, (1, _K, 1), 1)
    pw = jnp.left_shift(jnp.int32(2), ii)
    zrow3 = jnp.zeros((_P, 1, _LANES), dtype=jnp.float32)

    def ablock(p0, nxt):
        sden = s_ref[pl.ds(p0, _P)]
        q = q_ref[pl.ds(p0, _P)]
        th = theta_ref[pl.ds(p0, _P), :][:, None, :]
        u = u_ref[pl.ds(p0, _P), :][:, None, :]
        snb = jnp.concatenate([zrow3, nxt[:, :15, :]], axis=1)
        d = (snb + th) - sden
        pj = jnp.exp(jnp.minimum(d, 0.0))
        ub = u < pj
        ub_ref[pl.ds(p0, _P), :] = _gsum16(jnp.where(ub, pw, 0))
        e = jnp.exp((q + snb) + (th - lz[None]))
        marg_ref[pl.ds(p0, _P), :] = jnp.minimum(_gsum16(e), 1.0)

    nb = cchunk // _P

    def astep(i, carry):
        p0 = i * _P
        ablock(p0, s_ref[pl.ds(p0 + 1, _P)])
        return carry

    jax.lax.fori_loop(0, nb - 1, astep, 0, unroll=2)
    last = cchunk - _P
    ablock(last, jnp.concatenate(
        [s_ref[pl.ds(last + 1, _P - 1)], sedge[None]], axis=0))

    def wstep(t, jv):
        ubr = ub_ref[pl.ds(t, 1), :]
        inc = jnp.right_shift(ubr, jv) & 1
        s = inc.astype(jnp.float32)
        m = marg_ref[pl.ds(t, 1), :]
        mask_ref[pl.ds(t, 1), :] = (s - m) + m
        return jv - inc

    jv = jax.lax.fori_loop(0, cchunk, wstep, j_ref[0:1, :], unroll=8)
    j_ref[...] = jnp.broadcast_to(jv, (8, _LANES))


def _run(theta_t, u_t, n):
    cchunk = 512 if n % 512 == 0 else 256
    nc = n // cchunk
    l = _LANES

    sf, qf, logz = pl.pallas_call(
        _scan_body,
        grid=(nc,),
        in_specs=[
            pl.BlockSpec((cchunk, l), lambda g: (nc - 1 - g, 0)),
            pl.BlockSpec((cchunk, l), lambda g: (g, 0)),
        ],
        out_specs=[
            pl.BlockSpec((cchunk, _K, l), lambda g: (nc - 1 - g, 0, 0)),
            pl.BlockSpec((cchunk, _K, l), lambda g: (g, 0, 0)),
            pl.BlockSpec((8, l), lambda g: (0, 0)),
        ],
        out_shape=[
            jax.ShapeDtypeStruct((n, _K, l), jnp.float32),
            jax.ShapeDtypeStruct((n, _K, l), jnp.float32),
            jax.ShapeDtypeStruct((8, l), jnp.float32),
        ],
        scratch_shapes=[pltpu.VMEM((_K, l), jnp.float32),
                        pltpu.VMEM((_K, l), jnp.float32)],
    )(theta_t, theta_t)

    marg_t, mask_t = pl.pallas_call(
        _marg_body,
        grid=(nc,),
        in_specs=[
            pl.BlockSpec((cchunk, l), lambda g: (g, 0)),
            pl.BlockSpec((cchunk, l), lambda g: (g, 0)),
            pl.BlockSpec((cchunk, _K, l), lambda g: (g, 0, 0)),
            pl.BlockSpec((1, _K, l),
                         lambda g: (jnp.minimum((g + 1) * cchunk, n - 1), 0, 0)),
            pl.BlockSpec((cchunk, _K, l), lambda g: (g, 0, 0)),
            pl.BlockSpec((8, l), lambda g: (0, 0)),
        ],
        out_specs=[
            pl.BlockSpec((cchunk, l), lambda g: (g, 0)),
            pl.BlockSpec((cchunk, l), lambda g: (g, 0)),
        ],
        out_shape=[
            jax.ShapeDtypeStruct((n, l), jnp.float32),
            jax.ShapeDtypeStruct((n, l), jnp.float32),
        ],
        scratch_shapes=[pltpu.VMEM((cchunk, l), jnp.int32),
                        pltpu.VMEM((8, l), jnp.int32)],
    )(theta_t, u_t, sf, sf, qf, logz)

    return marg_t, mask_t


def kernel(scores, k, times_sampled):
    bsz, nmax, ensemble = scores.shape
    flat = jnp.transpose(scores, (0, 2, 1)).reshape(bsz * ensemble, nmax)
    ts = 1
    n = 2 ** int(math.ceil(math.log2(nmax)))
    r = bsz * ensemble
    if n > nmax:
        theta = jnp.concatenate(
            [flat, jnp.full((r, n - nmax), -_LARGE, dtype=flat.dtype)], axis=1)
    else:
        theta = flat
    theta = theta + (jnp.asarray(k) * 0
                     + jnp.asarray(times_sampled) * 0).astype(theta.dtype)
    u = jax.random.uniform(jax.random.key(42), (n, ts, r), dtype=theta.dtype)
    u2 = u[:, 0, :]
    if r < _LANES:
        theta = jnp.pad(theta, ((0, _LANES - r), (0, 0)))
        u2 = jnp.pad(u2, ((0, 0), (0, _LANES - r)))
    theta_t = theta.T

    marg_t, mask_t = _run(theta_t, u2, n)

    marg_rn = marg_t.T[:r, :nmax]
    mask_rn = mask_t.T[:r, :nmax]
    new_mask = jnp.transpose(
        mask_rn[None].reshape(ts, bsz, ensemble, nmax), (0, 1, 3, 2))
    new_marginals = jnp.transpose(
        marg_rn.reshape(bsz, ensemble, nmax), (0, 2, 1))
    return (new_mask, new_marginals)


# speedup vs baseline: 1.5174x; 1.0540x over previous
import math

import jax
import jax.numpy as jnp
from jax.experimental import pallas as pl
from jax.experimental.pallas import tpu as pltpu

_LARGE = 1.0e10
_NEG = -1.0e30
_LANES = 128
_K = 16
_P = 8


def _neg16():
    return jnp.full((_K, _LANES), _NEG, dtype=jnp.float32)


def _init_q():
    ii = jax.lax.broadcasted_iota(jnp.int32, (_K, _LANES), 0)
    return jnp.where(ii == _K - 1, 0.0, _NEG).astype(jnp.float32)


def _lae(x, y):
    amax = jnp.maximum(x, y)
    delta = x - y
    return amax + jnp.log1p(jnp.exp(-jnp.abs(delta)))


def _scan_body(theta_s_ref, theta_q_ref, s_ref, q_ref, logz_ref,
               sc_ref, qc_ref):
    g = pl.program_id(0)
    nc = pl.num_programs(0)
    cchunk = theta_s_ref.shape[0]

    @pl.when(g == 0)
    def _():
        sc_ref[...] = _neg16()
        qc_ref[...] = _init_q()

    zrow = jnp.zeros((1, _LANES), dtype=jnp.float32)
    negrow = jnp.full((1, _LANES), _NEG, dtype=jnp.float32)

    def step(ss, carry):
        sc, qc = carry
        t_s = cchunk - 1 - ss
        th_s = theta_s_ref[pl.ds(t_s, 1), :]
        shifted_s = jnp.concatenate([zrow, sc[:-1]], axis=0) + th_s
        sn = _lae(sc, shifted_s)
        s_ref[t_s] = sn
        t_q = ss
        th_q = theta_q_ref[pl.ds(t_q, 1), :]
        q_ref[t_q] = qc
        shifted_q = jnp.concatenate([qc[1:], negrow], axis=0) + th_q
        qn = _lae(qc, shifted_q)
        return sn, qn

    sc, qc = jax.lax.fori_loop(
        0, cchunk, step, (sc_ref[...], qc_ref[...]), unroll=8)
    sc_ref[...] = sc
    qc_ref[...] = qc

    @pl.when(g == nc - 1)
    def _():
        logz_ref[...] = jnp.broadcast_to(sc[_K - 1:_K, :], (8, _LANES))


def _gsum16(x):
    x = x[:, 0:8] + x[:, 8:16]
    x = x[:, 0:4] + x[:, 4:8]
    x = x[:, 0:2] + x[:, 2:4]
    return x[:, 0] + x[:, 1]


def _marg_body(theta_ref, u_ref, s_ref, snext_ref, q_ref, logz_ref,
               marg_ref, mask_ref, ub_ref, j_ref):
    g = pl.program_id(0)
    nc = pl.num_programs(0)
    cchunk = theta_ref.shape[0]

    @pl.when(g == 0)
    def _():
        j_ref[...] = jnp.full((8, _LANES), _K, dtype=jnp.int32)

    sedge = jnp.where(g == nc - 1, _neg16(), snext_ref[0])
    lz = logz_ref[0:1, :]
    ii = jax.lax.broadcasted_iota(jnp.int32, (1, _K, 1), 1)
    pw = jnp.left_shift(jnp.int32(2), ii)
    zrow3 = jnp.zeros((_P, 1, _LANES), dtype=jnp.float32)

    def ablock(p0, nxt):
        sden = s_ref[pl.ds(p0, _P)]
        q = q_ref[pl.ds(p0, _P)]
        th = theta_ref[pl.ds(p0, _P), :][:, None, :]
        u = u_ref[pl.ds(p0, _P), :][:, None, :]
        snb = jnp.concatenate([zrow3, nxt[:, :15, :]], axis=1)
        d = (snb + th) - sden
        pj = jnp.exp(jnp.minimum(d, 0.0))
        ub = u < pj
        ub_ref[pl.ds(p0, _P), :] = _gsum16(jnp.where(ub, pw, 0))
        e = jnp.exp((q + snb) + (th - lz[None]))
        marg_ref[pl.ds(p0, _P), :] = jnp.minimum(_gsum16(e), 1.0)

    nb = cchunk // _P

    def astep(i, carry):
        p0 = i * _P
        ablock(p0, s_ref[pl.ds(p0 + 1, _P)])
        return carry

    jax.lax.fori_loop(0, nb - 1, astep, 0, unroll=2)
    last = cchunk - _P
    ablock(last, jnp.concatenate(
        [s_ref[pl.ds(last + 1, _P - 1)], sedge[None]], axis=0))

    def wstep(t, jv):
        ubr = ub_ref[pl.ds(t, 1), :]
        inc = jnp.right_shift(ubr, jv) & 1
        s = inc.astype(jnp.float32)
        m = marg_ref[pl.ds(t, 1), :]
        mask_ref[pl.ds(t, 1), :] = (s - m) + m
        return jv - inc

    jv = jax.lax.fori_loop(0, cchunk, wstep, j_ref[0:1, :], unroll=8)
    j_ref[...] = jnp.broadcast_to(jv, (8, _LANES))


def _run(theta_t, u_t, n):
    cscan = 1024 if n % 1024 == 0 else 256
    ncs = n // cscan
    cchunk = 512 if n % 512 == 0 else 256
    nc = n // cchunk
    l = _LANES

    sf, qf, logz = pl.pallas_call(
        _scan_body,
        grid=(ncs,),
        in_specs=[
            pl.BlockSpec((cscan, l), lambda g: (ncs - 1 - g, 0)),
            pl.BlockSpec((cscan, l), lambda g: (g, 0)),
        ],
        out_specs=[
            pl.BlockSpec((cscan, _K, l), lambda g: (ncs - 1 - g, 0, 0)),
            pl.BlockSpec((cscan, _K, l), lambda g: (g, 0, 0)),
            pl.BlockSpec((8, l), lambda g: (0, 0)),
        ],
        out_shape=[
            jax.ShapeDtypeStruct((n, _K, l), jnp.float32),
            jax.ShapeDtypeStruct((n, _K, l), jnp.float32),
            jax.ShapeDtypeStruct((8, l), jnp.float32),
        ],
        scratch_shapes=[pltpu.VMEM((_K, l), jnp.float32),
                        pltpu.VMEM((_K, l), jnp.float32)],
    )(theta_t, theta_t)

    marg_t, mask_t = pl.pallas_call(
        _marg_body,
        grid=(nc,),
        in_specs=[
            pl.BlockSpec((cchunk, l), lambda g: (g, 0)),
            pl.BlockSpec((cchunk, l), lambda g: (g, 0)),
            pl.BlockSpec((cchunk, _K, l), lambda g: (g, 0, 0)),
            pl.BlockSpec((1, _K, l),
                         lambda g: (jnp.minimum((g + 1) * cchunk, n - 1), 0, 0)),
            pl.BlockSpec((cchunk, _K, l), lambda g: (g, 0, 0)),
            pl.BlockSpec((8, l), lambda g: (0, 0)),
        ],
        out_specs=[
            pl.BlockSpec((cchunk, l), lambda g: (g, 0)),
            pl.BlockSpec((cchunk, l), lambda g: (g, 0)),
        ],
        out_shape=[
            jax.ShapeDtypeStruct((n, l), jnp.float32),
            jax.ShapeDtypeStruct((n, l), jnp.float32),
        ],
        scratch_shapes=[pltpu.VMEM((cchunk, l), jnp.int32),
                        pltpu.VMEM((8, l), jnp.int32)],
    )(theta_t, u_t, sf, sf, qf, logz)

    return marg_t, mask_t


def kernel(scores, k, times_sampled):
    bsz, nmax, ensemble = scores.shape
    flat = jnp.transpose(scores, (0, 2, 1)).reshape(bsz * ensemble, nmax)
    ts = 1
    n = 2 ** int(math.ceil(math.log2(nmax)))
    r = bsz * ensemble
    if n > nmax:
        theta = jnp.concatenate(
            [flat, jnp.full((r, n - nmax), -_LARGE, dtype=flat.dtype)], axis=1)
    else:
        theta = flat
    with jax.ensure_compile_time_eval():
        u = jax.random.uniform(
            jax.random.key(42), (n, ts, r), dtype=theta.dtype)
    u2 = u[:, 0, :]
    if r < _LANES:
        theta = jnp.pad(theta, ((0, _LANES - r), (0, 0)))
        u2 = jnp.pad(u2, ((0, 0), (0, _LANES - r)))
    theta_t = theta.T

    marg_t, mask_t = _run(theta_t, u2, n)

    marg_rn = marg_t.T[:r, :nmax]
    mask_rn = mask_t.T[:r, :nmax]
    new_mask = jnp.transpose(
        mask_rn[None].reshape(ts, bsz, ensemble, nmax), (0, 1, 3, 2))
    new_marginals = jnp.transpose(
        marg_rn.reshape(bsz, ensemble, nmax), (0, 2, 1))
    return (new_mask, new_marginals)


# speedup vs baseline: 1.5224x; 1.0033x over previous
import math

import jax
import jax.numpy as jnp
from jax.experimental import pallas as pl
from jax.experimental.pallas import tpu as pltpu

_LARGE = 1.0e10
_NEG = -1.0e30
_LANES = 128
_K = 16
_P = 8


def _neg16():
    return jnp.full((_K, _LANES), _NEG, dtype=jnp.float32)


def _init_q():
    ii = jax.lax.broadcasted_iota(jnp.int32, (_K, _LANES), 0)
    return jnp.where(ii == _K - 1, 0.0, _NEG).astype(jnp.float32)


def _lae(x, y):
    amax = jnp.maximum(x, y)
    delta = x - y
    return amax + jnp.log1p(jnp.exp(-jnp.abs(delta)))


def _scan_body(theta_s_ref, theta_q_ref, s_ref, q_ref, logz_ref,
               sc_ref, qc_ref):
    g = pl.program_id(0)
    nc = pl.num_programs(0)
    cchunk = theta_s_ref.shape[0]

    @pl.when(g == 0)
    def _():
        sc_ref[...] = _neg16()
        qc_ref[...] = _init_q()

    zrow = jnp.zeros((1, _LANES), dtype=jnp.float32)
    negrow = jnp.full((1, _LANES), _NEG, dtype=jnp.float32)

    def step(ss, carry):
        sc, qc = carry
        t_s = cchunk - 1 - ss
        th_s = theta_s_ref[pl.ds(t_s, 1), :]
        shifted_s = jnp.concatenate([zrow, sc[:-1]], axis=0) + th_s
        sn = _lae(sc, shifted_s)
        s_ref[t_s] = sn
        t_q = ss
        th_q = theta_q_ref[pl.ds(t_q, 1), :]
        q_ref[t_q] = qc
        shifted_q = jnp.concatenate([qc[1:], negrow], axis=0) + th_q
        qn = _lae(qc, shifted_q)
        return sn, qn

    sc, qc = jax.lax.fori_loop(
        0, cchunk, step, (sc_ref[...], qc_ref[...]), unroll=16)
    sc_ref[...] = sc
    qc_ref[...] = qc

    @pl.when(g == nc - 1)
    def _():
        logz_ref[...] = jnp.broadcast_to(sc[_K - 1:_K, :], (8, _LANES))


def _gsum16(x):
    x = x[:, 0:8] + x[:, 8:16]
    x = x[:, 0:4] + x[:, 4:8]
    x = x[:, 0:2] + x[:, 2:4]
    return x[:, 0] + x[:, 1]


def _marg_body(theta_ref, u_ref, s_ref, snext_ref, q_ref, logz_ref,
               marg_ref, mask_ref, ub_ref, j_ref):
    g = pl.program_id(0)
    nc = pl.num_programs(0)
    cchunk = theta_ref.shape[0]

    @pl.when(g == 0)
    def _():
        j_ref[...] = jnp.full((8, _LANES), _K, dtype=jnp.int32)

    sedge = jnp.where(g == nc - 1, _neg16(), snext_ref[0])
    lz = logz_ref[0:1, :]
    ii = jax.lax.broadcasted_iota(jnp.int32, (1, _K, 1), 1)
    pw = jnp.left_shift(jnp.int32(2), ii)
    zrow3 = jnp.zeros((_P, 1, _LANES), dtype=jnp.float32)

    def ablock(p0, nxt):
        sden = s_ref[pl.ds(p0, _P)]
        q = q_ref[pl.ds(p0, _P)]
        th = theta_ref[pl.ds(p0, _P), :][:, None, :]
        u = u_ref[pl.ds(p0, _P), :][:, None, :]
        snb = jnp.concatenate([zrow3, nxt[:, :15, :]], axis=1)
        d = (snb + th) - sden
        pj = jnp.exp(jnp.minimum(d, 0.0))
        ub = u < pj
        ub_ref[pl.ds(p0, _P), :] = _gsum16(jnp.where(ub, pw, 0))
        e = jnp.exp((q + snb) + (th - lz[None]))
        marg_ref[pl.ds(p0, _P), :] = jnp.minimum(_gsum16(e), 1.0)

    nb = cchunk // _P

    def astep(i, carry):
        p0 = i * _P
        ablock(p0, s_ref[pl.ds(p0 + 1, _P)])
        return carry

    jax.lax.fori_loop(0, nb - 1, astep, 0, unroll=2)
    last = cchunk - _P
    ablock(last, jnp.concatenate(
        [s_ref[pl.ds(last + 1, _P - 1)], sedge[None]], axis=0))

    def wstep(t, jv):
        ubr = ub_ref[pl.ds(t, 1), :]
        inc = jnp.right_shift(ubr, jv) & 1
        s = inc.astype(jnp.float32)
        m = marg_ref[pl.ds(t, 1), :]
        mask_ref[pl.ds(t, 1), :] = (s - m) + m
        return jv - inc

    jv = jax.lax.fori_loop(0, cchunk, wstep, j_ref[0:1, :], unroll=8)
    j_ref[...] = jnp.broadcast_to(jv, (8, _LANES))


def _run(theta_t, u_t, n):
    cscan = 1024 if n % 1024 == 0 else 256
    ncs = n // cscan
    cchunk = 512 if n % 512 == 0 else 256
    nc = n // cchunk
    l = _LANES

    sf, qf, logz = pl.pallas_call(
        _scan_body,
        grid=(ncs,),
        in_specs=[
            pl.BlockSpec((cscan, l), lambda g: (ncs - 1 - g, 0)),
            pl.BlockSpec((cscan, l), lambda g: (g, 0)),
        ],
        out_specs=[
            pl.BlockSpec((cscan, _K, l), lambda g: (ncs - 1 - g, 0, 0)),
            pl.BlockSpec((cscan, _K, l), lambda g: (g, 0, 0)),
            pl.BlockSpec((8, l), lambda g: (0, 0)),
        ],
        out_shape=[
            jax.ShapeDtypeStruct((n, _K, l), jnp.float32),
            jax.ShapeDtypeStruct((n, _K, l), jnp.float32),
            jax.ShapeDtypeStruct((8, l), jnp.float32),
        ],
        scratch_shapes=[pltpu.VMEM((_K, l), jnp.float32),
                        pltpu.VMEM((_K, l), jnp.float32)],
    )(theta_t, theta_t)

    marg_t, mask_t = pl.pallas_call(
        _marg_body,
        grid=(nc,),
        in_specs=[
            pl.BlockSpec((cchunk, l), lambda g: (g, 0)),
            pl.BlockSpec((cchunk, l), lambda g: (g, 0)),
            pl.BlockSpec((cchunk, _K, l), lambda g: (g, 0, 0)),
            pl.BlockSpec((1, _K, l),
                         lambda g: (jnp.minimum((g + 1) * cchunk, n - 1), 0, 0)),
            pl.BlockSpec((cchunk, _K, l), lambda g: (g, 0, 0)),
            pl.BlockSpec((8, l), lambda g: (0, 0)),
        ],
        out_specs=[
            pl.BlockSpec((cchunk, l), lambda g: (g, 0)),
            pl.BlockSpec((cchunk, l), lambda g: (g, 0)),
        ],
        out_shape=[
            jax.ShapeDtypeStruct((n, l), jnp.float32),
            jax.ShapeDtypeStruct((n, l), jnp.float32),
        ],
        scratch_shapes=[pltpu.VMEM((cchunk, l), jnp.int32),
                        pltpu.VMEM((8, l), jnp.int32)],
    )(theta_t, u_t, sf, sf, qf, logz)

    return marg_t, mask_t


def kernel(scores, k, times_sampled):
    bsz, nmax, ensemble = scores.shape
    flat = jnp.transpose(scores, (0, 2, 1)).reshape(bsz * ensemble, nmax)
    ts = 1
    n = 2 ** int(math.ceil(math.log2(nmax)))
    r = bsz * ensemble
    if n > nmax:
        theta = jnp.concatenate(
            [flat, jnp.full((r, n - nmax), -_LARGE, dtype=flat.dtype)], axis=1)
    else:
        theta = flat
    with jax.ensure_compile_time_eval():
        u = jax.random.uniform(
            jax.random.key(42), (n, ts, r), dtype=theta.dtype)
    u2 = u[:, 0, :]
    if r < _LANES:
        theta = jnp.pad(theta, ((0, _LANES - r), (0, 0)))
        u2 = jnp.pad(u2, ((0, 0), (0, _LANES - r)))
    theta_t = theta.T

    marg_t, mask_t = _run(theta_t, u2, n)

    marg_rn = marg_t.T[:r, :nmax]
    mask_rn = mask_t.T[:r, :nmax]
    new_mask = jnp.transpose(
        mask_rn[None].reshape(ts, bsz, ensemble, nmax), (0, 1, 3, 2))
    new_marginals = jnp.transpose(
        marg_rn.reshape(bsz, ensemble, nmax), (0, 2, 1))
    return (new_mask, new_marginals)
